# Initial kernel scaffold; baseline (speedup 1.0000x reference)
import jax
import jax.numpy as jnp
from jax.experimental import pallas as pl


def kernel(features, centers):
    raise NotImplementedError("write your pallas kernel here")


# speedup vs baseline: 1.5885x; 1.5885x over previous
import jax
import jax.numpy as jnp
from jax.experimental import pallas as pl
from jax.experimental.pallas import tpu as pltpu


def _tc_body(f_ref, ct_ref, out_ref):
    i = pl.program_id(0)
    nsteps = pl.num_programs(0)
    f = f_ref[...]
    ct = ct_ref[...]
    csq = jnp.sum(ct * ct, axis=0, keepdims=True)
    dot = jnp.dot(f, ct, preferred_element_type=jnp.float32)
    g = csq - 2.0 * dot
    min_g = jnp.min(g, axis=1, keepdims=True)
    fsq = jnp.sum(f * f, axis=1, keepdims=True)
    sq = jnp.maximum(fsq + min_g, 0.0)
    dist = jnp.minimum(jnp.sqrt(sq), 1000000.0)
    part = jnp.sum(dist)

    @pl.when(i == 0)
    def _():
        out_ref[0, 0] = 0.0

    out_ref[0, 0] += part

    @pl.when(i == nsteps - 1)
    def _():
        out_ref[0, 0] = out_ref[0, 0] * (1.0 / (nsteps * f.shape[0]))


def kernel(features, centers):
    n, d = features.shape
    k = centers.shape[0]
    bn = 2048
    ct = centers.T

    out = pl.pallas_call(
        _tc_body,
        grid=(n // bn,),
        in_specs=[
            pl.BlockSpec((bn, d), lambda i: (i, 0)),
            pl.BlockSpec((d, k), lambda i: (0, 0)),
        ],
        out_specs=pl.BlockSpec((1, 1), lambda i: (0, 0),
                               memory_space=pltpu.SMEM),
        out_shape=jax.ShapeDtypeStruct((1, 1), jnp.float32),
    )(features, ct)
    return out[0, 0]


# speedup vs baseline: 2.3236x; 1.4627x over previous
import jax
import jax.numpy as jnp
from jax.experimental import pallas as pl
from jax.experimental.pallas import tpu as pltpu


def _tc_body(ft_ref, c_ref, out_ref, caug_ref):
    i = pl.program_id(0)
    nsteps = pl.num_programs(0)

    @pl.when(i == 0)
    def _():
        c = c_ref[...]
        csq = jnp.sum(c * c, axis=1, keepdims=True)
        ones = jnp.ones((c.shape[0], 1), jnp.float32)
        caug_ref[...] = jnp.concatenate([c * -2.0, csq, ones], axis=1)
        out_ref[0, 0] = 0.0

    ft = ft_ref[...]
    fsq = jnp.sum(ft * ft, axis=0, keepdims=True)
    ones_r = jnp.ones((1, ft.shape[1]), jnp.float32)
    faug = jnp.concatenate([ft, ones_r, fsq], axis=0)
    sq = jax.lax.dot_general(
        caug_ref[...], faug, (((1,), (0,)), ((), ())),
        preferred_element_type=jnp.float32)
    minsq = jnp.min(sq, axis=0, keepdims=True)
    dist = jnp.minimum(jnp.sqrt(jnp.maximum(minsq, 0.0)), 1000000.0)
    out_ref[0, 0] += jnp.sum(dist)

    @pl.when(i == nsteps - 1)
    def _():
        out_ref[0, 0] = out_ref[0, 0] * (1.0 / (nsteps * ft.shape[1]))


def kernel(features, centers):
    n, d = features.shape
    k = centers.shape[0]
    bn = 2048
    ft = features.T

    out = pl.pallas_call(
        _tc_body,
        grid=(n // bn,),
        in_specs=[
            pl.BlockSpec((d, bn), lambda i: (0, i)),
            pl.BlockSpec((k, d), lambda i: (0, 0)),
        ],
        out_specs=pl.BlockSpec((1, 1), lambda i: (0, 0),
                               memory_space=pltpu.SMEM),
        out_shape=jax.ShapeDtypeStruct((1, 1), jnp.float32),
        scratch_shapes=[pltpu.VMEM((k, d + 2), jnp.float32)],
    )(ft, centers)
    return out[0, 0]
